# baseline (device time: 23403 ns/iter reference)
import jax
import jax.numpy as jnp
from jax import lax
from jax.experimental import pallas as pl
from jax.experimental.pallas import tpu as pltpu

M, N = 512, 512
ROWS = 64
SUBS = 4
SR = ROWS // SUBS


def kernel(x):
    x2 = x.reshape(M, N)

    def body(x_ref, out_ref, acc_ref, recv_buf, send_sems, recv_sems):
        my_x = lax.axis_index("x")
        my_y = lax.axis_index("y")
        my_z = lax.axis_index("z")

        l1 = [my_z, my_y]
        l2 = [my_y, my_z]
        base = [0, 256]
        p_x = (1 - my_x, my_y, my_z)

        def peer_l1(h, c):
            return (my_x, my_y, c) if h == 0 else (my_x, c, my_z)

        def peer_l2(h, c):
            return (my_x, c, my_z) if h == 0 else (my_x, my_y, c)

        barrier_sem = pltpu.get_barrier_semaphore()
        peers = [(my_x, my_y, (my_z + d) % 4) for d in (1, 2, 3)]
        peers += [(my_x, (my_y + d) % 4, my_z) for d in (1, 2, 3)]
        peers += [p_x]
        for p in peers:
            pl.semaphore_signal(
                barrier_sem, inc=1, device_id=p,
                device_id_type=pl.DeviceIdType.MESH,
            )
        pl.semaphore_wait(barrier_sem, 7)

        acc_ref[...] = x_ref[...].astype(jnp.bfloat16)

        rdmas = {}

        def own_off(h):
            return base[h] + l1[h] * ROWS

        def a2a_start(h, stage, sub, src_off_fn, dst_acc):
            coord = l1[h] if stage in (0, 3) else l2[h]
            for d in (1, 2, 3):
                pc = (coord + d) % 4
                p = peer_l1(h, pc) if stage in (0, 3) else peer_l2(h, pc)
                soff = src_off_fn(pc) + sub * SR
                src = acc_ref.at[pl.ds(soff, SR), :]
                if dst_acc:
                    dst = acc_ref.at[pl.ds(soff, SR), :]
                else:
                    slot = 3 * (stage == 1) + (d - 1)
                    dst = recv_buf.at[h, slot, pl.ds(sub * SR, SR), :]
                r = pltpu.make_async_remote_copy(
                    src_ref=src,
                    dst_ref=dst,
                    send_sem=send_sems.at[h, stage, d - 1, sub],
                    recv_sem=recv_sems.at[h, stage, d - 1, sub],
                    device_id=p,
                    device_id_type=pl.DeviceIdType.MESH,
                )
                r.start()
                rdmas[(h, stage, d, sub)] = r

        def add_slots(h, first_slot, sub):
            rows = pl.ds(own_off(h) + sub * SR, SR)
            srows = pl.ds(sub * SR, SR)
            acc_ref[rows, :] += (
                recv_buf[h, first_slot, srows, :]
                + recv_buf[h, first_slot + 1, srows, :]
                + recv_buf[h, first_slot + 2, srows, :]
            )

        for h in (0, 1):
            for sub in range(SUBS):
                a2a_start(h, 0, sub, lambda pc, h=h: base[h] + pc * ROWS, False)

        for sub in range(SUBS):
            for h in (0, 1):
                for d in (1, 2, 3):
                    rdmas[(h, 0, d, sub)].wait_recv()
                add_slots(h, 0, sub)
                a2a_start(h, 1, sub, lambda pc, h=h: own_off(h), False)

        for sub in range(SUBS):
            for h in (0, 1):
                for d in (1, 2, 3):
                    rdmas[(h, 1, d, sub)].wait_recv()
                    rdmas[(h, 1, d, sub)].wait_send()
                add_slots(h, 3, sub)
                r = pltpu.make_async_remote_copy(
                    src_ref=acc_ref.at[pl.ds(own_off(h) + sub * SR, SR), :],
                    dst_ref=recv_buf.at[h, 6, pl.ds(sub * SR, SR), :],
                    send_sem=send_sems.at[h, 2, 0, sub],
                    recv_sem=recv_sems.at[h, 2, 0, sub],
                    device_id=p_x,
                    device_id_type=pl.DeviceIdType.MESH,
                )
                r.start()
                rdmas[(h, 2, 1, sub)] = r

        for sub in range(SUBS):
            for h in (0, 1):
                rdmas[(h, 2, 1, sub)].wait_recv()
                rdmas[(h, 2, 1, sub)].wait_send()
                rows = pl.ds(own_off(h) + sub * SR, SR)
                acc_ref[rows, :] += recv_buf[h, 6, pl.ds(sub * SR, SR), :]
                a2a_start(h, 3, sub, lambda pc, h=h: own_off(h), True)

        for sub in range(SUBS):
            for h in (0, 1):
                for d in (1, 2, 3):
                    rdmas[(h, 3, d, sub)].wait_recv()

        out_ref[...] = acc_ref[...].astype(jnp.float32)

        for sub in range(SUBS):
            for h in (0, 1):
                for d in (1, 2, 3):
                    rdmas[(h, 0, d, sub)].wait_send()
                    rdmas[(h, 3, d, sub)].wait_send()

    return pl.pallas_call(
        body,
        out_shape=jax.ShapeDtypeStruct((M, N), jnp.float32),
        in_specs=[pl.BlockSpec(memory_space=pltpu.VMEM)],
        out_specs=pl.BlockSpec(memory_space=pltpu.VMEM),
        scratch_shapes=[
            pltpu.VMEM((M, N), jnp.bfloat16),
            pltpu.VMEM((2, 7, ROWS, N), jnp.bfloat16),
            pltpu.SemaphoreType.DMA((2, 4, 3, SUBS)),
            pltpu.SemaphoreType.DMA((2, 4, 3, SUBS)),
        ],
        compiler_params=pltpu.CompilerParams(collective_id=0),
    )(x2)
